# baseline (speedup 1.0000x reference)
import functools

import jax
import jax.numpy as jnp
from jax import lax
from jax.experimental import pallas as pl
from jax.experimental.pallas import tpu as pltpu
from jax.experimental.pallas import tpu_sc as plsc

D_MODEL = 1024
N_ROWS = 8192
N_WORKERS = 32
ROWS_PER_WORKER = N_ROWS // N_WORKERS
CHUNK = 32
N_CHUNKS = ROWS_PER_WORKER // CHUNK
VECS_PER_ROW = D_MODEL // 16

_mesh = plsc.VectorSubcoreMesh(core_axis_name="c", subcore_axis_name="s")


@functools.partial(
    pl.kernel,
    mesh=_mesh,
    out_type=jax.ShapeDtypeStruct((N_ROWS, D_MODEL), jnp.float32),
    scratch_types=[
        pltpu.VMEM((CHUNK,), jnp.int32),
        pltpu.VMEM((CHUNK, D_MODEL), jnp.float32),
        pltpu.VMEM((CHUNK, D_MODEL), jnp.float32),
        pltpu.SemaphoreType.DMA,
        pltpu.SemaphoreType.DMA,
        pltpu.SemaphoreType.DMA,
    ],
)
def _lookup_add(x_hbm, ids_hbm, pe_hbm, out_hbm,
                idx_v, x_v, pe_v, sem_x, sem_pe, sem_out):
    wid = lax.axis_index("s") * 2 + lax.axis_index("c")
    base = wid * ROWS_PER_WORKER

    def chunk_body(ch, _):
        row0 = base + ch * CHUNK
        pltpu.sync_copy(ids_hbm.at[pl.ds(row0, CHUNK)], idx_v)
        cx = pltpu.async_copy(x_hbm.at[pl.ds(row0, CHUNK), :], x_v, sem_x)
        cp = pltpu.async_copy(pe_hbm.at[idx_v], pe_v, sem_pe)
        cx.wait()
        cp.wait()

        def row_body(r, _):
            def vec_body(j, _):
                sl = pl.ds(j * 16, 16)
                x_v[r, sl] = x_v[r, sl] + pe_v[r, sl]
                return ()
            lax.fori_loop(0, VECS_PER_ROW, vec_body, (), unroll=8)
            return ()
        lax.fori_loop(0, CHUNK, row_body, ())

        pltpu.async_copy(x_v, out_hbm.at[pl.ds(row0, CHUNK), :], sem_out).wait()
        return ()

    lax.fori_loop(0, N_CHUNKS, chunk_body, ())


def kernel(x, position_ids, pe_weight):
    B, L, D = x.shape
    xf = x.reshape(B * L, D)
    ids = position_ids.reshape(B * L).astype(jnp.int32)
    out = _lookup_add(xf, ids, pe_weight)
    return out.reshape(B, L, D)


# speedup vs baseline: 1.1566x; 1.1566x over previous
import jax
import jax.numpy as jnp
from jax import lax
from jax.experimental import pallas as pl
from jax.experimental.pallas import tpu as pltpu
from jax.experimental.pallas import tpu_sc as plsc

D_MODEL = 1024
N_ROWS = 8192
N_WORKERS = 32
ROWS_PER_WORKER = N_ROWS // N_WORKERS
CHUNK = 16
N_CHUNKS = ROWS_PER_WORKER // CHUNK
NBUF = 2
VECS_PER_ROW = D_MODEL // 16

_mesh = plsc.VectorSubcoreMesh(core_axis_name="c", subcore_axis_name="s")

_scratch = []
for _ in range(NBUF):
    _scratch += [
        pltpu.VMEM((CHUNK,), jnp.int32),
        pltpu.VMEM((CHUNK, D_MODEL), jnp.float32),
        pltpu.VMEM((CHUNK, D_MODEL), jnp.float32),
        pltpu.VMEM((CHUNK, D_MODEL), jnp.float32),
        pltpu.SemaphoreType.DMA,
        pltpu.SemaphoreType.DMA,
        pltpu.SemaphoreType.DMA,
    ]


@pl.kernel(
    mesh=_mesh,
    out_type=jax.ShapeDtypeStruct((N_ROWS, D_MODEL), jnp.float32),
    scratch_types=_scratch,
)
def _lookup_add(x_hbm, ids_hbm, pe_hbm, out_hbm, *bufs):
    idx_b = [bufs[7 * b + 0] for b in range(NBUF)]
    x_b = [bufs[7 * b + 1] for b in range(NBUF)]
    pe_b = [bufs[7 * b + 2] for b in range(NBUF)]
    o_b = [bufs[7 * b + 3] for b in range(NBUF)]
    sx = [bufs[7 * b + 4] for b in range(NBUF)]
    sp = [bufs[7 * b + 5] for b in range(NBUF)]
    so = [bufs[7 * b + 6] for b in range(NBUF)]

    wid = lax.axis_index("s") * 2 + lax.axis_index("c")
    base = wid * ROWS_PER_WORKER

    def start_load(ch, b):
        row0 = base + ch * CHUNK
        pltpu.sync_copy(ids_hbm.at[pl.ds(row0, CHUNK)], idx_b[b])
        hx = pltpu.async_copy(x_hbm.at[pl.ds(row0, CHUNK), :], x_b[b], sx[b])
        hp = pltpu.async_copy(pe_hbm.at[idx_b[b]], pe_b[b], sp[b])
        return hx, hp

    loads = {}
    writes = {}
    for ch in range(NBUF):
        loads[ch] = start_load(ch, ch % NBUF)

    for ch in range(N_CHUNKS):
        b = ch % NBUF
        hx, hp = loads.pop(ch)
        hx.wait()
        hp.wait()
        if ch >= NBUF:
            writes.pop(ch - NBUF).wait()

        def row_body(r, _):
            def vec_body(j, _):
                sl = pl.ds(j * 16, 16)
                o_b[b][r, sl] = x_b[b][r, sl] + pe_b[b][r, sl]
                return ()
            lax.fori_loop(0, VECS_PER_ROW, vec_body, (), unroll=8)
            return ()
        lax.fori_loop(0, CHUNK, row_body, ())

        row0 = base + ch * CHUNK
        writes[ch] = pltpu.async_copy(
            o_b[b], out_hbm.at[pl.ds(row0, CHUNK), :], so[b])
        if ch + NBUF < N_CHUNKS:
            loads[ch + NBUF] = start_load(ch + NBUF, b)

    for ch in sorted(writes):
        writes.pop(ch).wait()


def kernel(x, position_ids, pe_weight):
    B, L, D = x.shape
    xf = x.reshape(B * L, D)
    ids = position_ids.reshape(B * L).astype(jnp.int32)
    out = _lookup_add(xf, ids, pe_weight)
    return out.reshape(B, L, D)


# speedup vs baseline: 2.3301x; 2.0146x over previous
import jax
import jax.numpy as jnp
from jax import lax
from jax.experimental import pallas as pl
from jax.experimental.pallas import tpu as pltpu
from jax.experimental.pallas import tpu_sc as plsc

D_MODEL = 1024
N_ROWS = 8192
N_WORKERS = 32
ROWS_PER_WORKER = N_ROWS // N_WORKERS
CHUNK = 16
N_CHUNKS = ROWS_PER_WORKER // CHUNK
NBUF = 2
VECS_PER_ROW = D_MODEL // 16

_mesh = plsc.VectorSubcoreMesh(core_axis_name="c", subcore_axis_name="s")

_scratch = []
for _ in range(NBUF):
    _scratch += [
        pltpu.VMEM((CHUNK,), jnp.int32),
        pltpu.VMEM((CHUNK, D_MODEL), jnp.float32),
        pltpu.VMEM((CHUNK, D_MODEL), jnp.float32),
        pltpu.VMEM((CHUNK, D_MODEL), jnp.float32),
        pltpu.SemaphoreType.DMA,
        pltpu.SemaphoreType.DMA,
        pltpu.SemaphoreType.DMA,
    ]


@pl.kernel(
    mesh=_mesh,
    out_type=jax.ShapeDtypeStruct((N_ROWS, D_MODEL), jnp.float32),
    scratch_types=_scratch,
)
def _lookup_add(x_hbm, ids_hbm, pe_hbm, out_hbm, *bufs):
    idx_b = [bufs[7 * b + 0] for b in range(NBUF)]
    x_b = [bufs[7 * b + 1] for b in range(NBUF)]
    pe_b = [bufs[7 * b + 2] for b in range(NBUF)]
    o_b = [bufs[7 * b + 3] for b in range(NBUF)]
    sx = [bufs[7 * b + 4] for b in range(NBUF)]
    sp = [bufs[7 * b + 5] for b in range(NBUF)]
    so = [bufs[7 * b + 6] for b in range(NBUF)]

    wid = lax.axis_index("s") * 2 + lax.axis_index("c")
    base = wid * ROWS_PER_WORKER

    def start_load(ch, b):
        row0 = base + ch * CHUNK
        pltpu.sync_copy(ids_hbm.at[pl.ds(row0, CHUNK)], idx_b[b])
        hx = pltpu.async_copy(x_hbm.at[pl.ds(row0, CHUNK), :], x_b[b], sx[b])
        hp = pltpu.async_copy(pe_hbm.at[idx_b[b]], pe_b[b], sp[b])
        return hx, hp

    loads = {}
    writes = {}
    for ch in range(NBUF):
        loads[ch] = start_load(ch, ch % NBUF)

    for ch in range(N_CHUNKS):
        b = ch % NBUF
        hx, hp = loads.pop(ch)
        hx.wait()
        hp.wait()
        if ch >= NBUF:
            writes.pop(ch - NBUF).wait()

        xb, pb, ob = x_b[b], pe_b[b], o_b[b]

        @plsc.parallel_loop(0, CHUNK * VECS_PER_ROW, unroll=8)
        def _add(t):
            r = t // VECS_PER_ROW
            j = t % VECS_PER_ROW
            sl = pl.ds(j * 16, 16)
            ob[r, sl] = xb[r, sl] + pb[r, sl]

        row0 = base + ch * CHUNK
        writes[ch] = pltpu.async_copy(
            o_b[b], out_hbm.at[pl.ds(row0, CHUNK), :], so[b])
        if ch + NBUF < N_CHUNKS:
            loads[ch + NBUF] = start_load(ch + NBUF, b)

    for ch in sorted(writes):
        writes.pop(ch).wait()


def kernel(x, position_ids, pe_weight):
    B, L, D = x.shape
    xf = x.reshape(B * L, D)
    ids = position_ids.reshape(B * L).astype(jnp.int32)
    out = _lookup_add(xf, ids, pe_weight)
    return out.reshape(B, L, D)


# speedup vs baseline: 2.3471x; 1.0073x over previous
import jax
import jax.numpy as jnp
from jax import lax
from jax.experimental import pallas as pl
from jax.experimental.pallas import tpu as pltpu
from jax.experimental.pallas import tpu_sc as plsc

D_MODEL = 1024
N_ROWS = 8192
N_WORKERS = 32
ROWS_PER_WORKER = N_ROWS // N_WORKERS
CHUNK = 16
N_CHUNKS = ROWS_PER_WORKER // CHUNK
NBUF = 3
VECS_PER_ROW = D_MODEL // 16

_mesh = plsc.VectorSubcoreMesh(core_axis_name="c", subcore_axis_name="s")

_scratch = []
for _ in range(NBUF):
    _scratch += [
        pltpu.VMEM((CHUNK,), jnp.int32),
        pltpu.VMEM((CHUNK, D_MODEL), jnp.float32),
        pltpu.VMEM((CHUNK, D_MODEL), jnp.float32),
        pltpu.SemaphoreType.DMA,
        pltpu.SemaphoreType.DMA,
        pltpu.SemaphoreType.DMA,
    ]


@pl.kernel(
    mesh=_mesh,
    out_type=jax.ShapeDtypeStruct((N_ROWS, D_MODEL), jnp.float32),
    scratch_types=_scratch,
)
def _lookup_add(x_hbm, ids_hbm, pe_hbm, out_hbm, *bufs):
    idx_b = [bufs[6 * b + 0] for b in range(NBUF)]
    x_b = [bufs[6 * b + 1] for b in range(NBUF)]
    pe_b = [bufs[6 * b + 2] for b in range(NBUF)]
    sx = [bufs[6 * b + 3] for b in range(NBUF)]
    sp = [bufs[6 * b + 4] for b in range(NBUF)]
    so = [bufs[6 * b + 5] for b in range(NBUF)]

    wid = lax.axis_index("s") * 2 + lax.axis_index("c")
    base = wid * ROWS_PER_WORKER

    def start_load(ch):
        b = ch % NBUF
        row0 = base + ch * CHUNK
        pltpu.sync_copy(ids_hbm.at[pl.ds(row0, CHUNK)], idx_b[b])
        hx = pltpu.async_copy(x_hbm.at[pl.ds(row0, CHUNK), :], x_b[b], sx[b])
        hp = pltpu.async_copy(pe_hbm.at[idx_b[b]], pe_b[b], sp[b])
        return hx, hp

    loads = {}
    writes = {}
    for ch in range(NBUF - 1):
        loads[ch] = start_load(ch)

    for ch in range(N_CHUNKS):
        b = ch % NBUF
        hx, hp = loads.pop(ch)
        hx.wait()
        hp.wait()

        xb, pb = x_b[b], pe_b[b]

        @plsc.parallel_loop(0, CHUNK * VECS_PER_ROW, unroll=8)
        def _add(t):
            r = t // VECS_PER_ROW
            j = t % VECS_PER_ROW
            sl = pl.ds(j * 16, 16)
            plsc.addupdate(xb.at[r, sl], pb[r, sl])

        row0 = base + ch * CHUNK
        writes[ch] = pltpu.async_copy(
            xb, out_hbm.at[pl.ds(row0, CHUNK), :], so[b])
        nxt = ch + NBUF - 1
        if nxt < N_CHUNKS:
            if nxt >= NBUF:
                writes.pop(nxt - NBUF).wait()
            loads[nxt] = start_load(nxt)

    for ch in sorted(writes):
        writes.pop(ch).wait()


def kernel(x, position_ids, pe_weight):
    B, L, D = x.shape
    xf = x.reshape(B * L, D)
    ids = position_ids.reshape(B * L).astype(jnp.int32)
    out = _lookup_add(xf, ids, pe_weight)
    return out.reshape(B, L, D)


# speedup vs baseline: 2.3696x; 1.0096x over previous
import jax
import jax.numpy as jnp
from jax import lax
from jax.experimental import pallas as pl
from jax.experimental.pallas import tpu as pltpu
from jax.experimental.pallas import tpu_sc as plsc

D_MODEL = 1024
N_ROWS = 8192
N_WORKERS = 32
ROWS_PER_WORKER = N_ROWS // N_WORKERS
CHUNK = 16
N_CHUNKS = ROWS_PER_WORKER // CHUNK
NXB = 4
NPB = 3
LEAD = 2
VECS_PER_ROW = D_MODEL // 16

_mesh = plsc.VectorSubcoreMesh(core_axis_name="c", subcore_axis_name="s")

_scratch = (
    [pltpu.VMEM((ROWS_PER_WORKER,), jnp.int32)]
    + [pltpu.VMEM((CHUNK, D_MODEL), jnp.float32) for _ in range(NXB + NPB)]
    + [pltpu.SemaphoreType.DMA for _ in range(2 * NXB + NPB)]
)


@pl.kernel(
    mesh=_mesh,
    out_type=jax.ShapeDtypeStruct((N_ROWS, D_MODEL), jnp.float32),
    scratch_types=_scratch,
)
def _lookup_add(x_hbm, ids_hbm, pe_hbm, out_hbm, idx_all, *bufs):
    x_b = list(bufs[:NXB])
    pe_b = list(bufs[NXB:NXB + NPB])
    sems = bufs[NXB + NPB:]
    sx = list(sems[:NXB])
    so = list(sems[NXB:2 * NXB])
    sp = list(sems[2 * NXB:])

    wid = lax.axis_index("s") * 2 + lax.axis_index("c")
    base = wid * ROWS_PER_WORKER

    pltpu.sync_copy(ids_hbm.at[pl.ds(base, ROWS_PER_WORKER)], idx_all)

    def start_load(ch):
        row0 = base + ch * CHUNK
        hx = pltpu.async_copy(
            x_hbm.at[pl.ds(row0, CHUNK), :], x_b[ch % NXB], sx[ch % NXB])
        hp = pltpu.async_copy(
            pe_hbm.at[idx_all.at[pl.ds(ch * CHUNK, CHUNK)]],
            pe_b[ch % NPB], sp[ch % NPB])
        return hx, hp

    loads = {}
    writes = {}
    for ch in range(LEAD):
        loads[ch] = start_load(ch)

    for ch in range(N_CHUNKS):
        hx, hp = loads.pop(ch)
        hx.wait()
        hp.wait()

        xb, pb = x_b[ch % NXB], pe_b[ch % NPB]

        @plsc.parallel_loop(0, CHUNK * VECS_PER_ROW, unroll=8)
        def _add(t):
            r = t // VECS_PER_ROW
            j = t % VECS_PER_ROW
            sl = pl.ds(j * 16, 16)
            plsc.addupdate(xb.at[r, sl], pb[r, sl])

        row0 = base + ch * CHUNK
        writes[ch] = pltpu.async_copy(
            xb, out_hbm.at[pl.ds(row0, CHUNK), :], so[ch % NXB])
        nxt = ch + LEAD
        if nxt < N_CHUNKS:
            if nxt >= NXB:
                writes.pop(nxt - NXB).wait()
            loads[nxt] = start_load(nxt)

    for ch in sorted(writes):
        writes.pop(ch).wait()


def kernel(x, position_ids, pe_weight):
    B, L, D = x.shape
    xf = x.reshape(B * L, D)
    ids = position_ids.reshape(B * L).astype(jnp.int32)
    out = _lookup_add(xf, ids, pe_weight)
    return out.reshape(B, L, D)
